# baseline (speedup 1.0000x reference)
import jax
import jax.numpy as jnp
from jax.experimental import pallas as pl
from jax.experimental.pallas import tpu as pltpu

_B = 1024
_L = 200
_N = 1000
_BM = 200
_NB = _N // _BM


def _infectivity_body(ti_ref, tjs_ref, hist_hbm, emb_hbm, out_ref,
                      gt, hist_i, hist_f, emb_v, hist_sem, emb_sems):
    step = pl.program_id(0)

    def emb_cp(i):
        blk = pl.ds(i * _BM, _BM)
        return pltpu.make_async_copy(emb_hbm.at[blk, :], emb_v.at[blk, :],
                                     emb_sems.at[i])

    @pl.when(step == 0)
    def _prologue():
        hist_cp = pltpu.make_async_copy(hist_hbm, hist_i, hist_sem)
        hist_cp.start()
        for i in range(_NB):
            emb_cp(i).start()
        gt[...] = jnp.exp(tjs_ref[...] - ti_ref[...])
        hist_cp.wait()
        hist_f[...] = hist_i[...].astype(jnp.float32)

    blk = pl.ds(step * _BM, _BM)
    pltpu.make_async_copy(emb_hbm.at[blk, :], emb_v.at[blk, :],
                          emb_sems.at[step]).wait()
    a = jax.lax.dot_general(
        emb_v[blk, :], hist_f[...], (((1,), (1,)), ((), ())),
        preferred_element_type=jnp.float32)
    out_ref[...] = jax.lax.dot_general(
        a, gt[...], (((1,), (1,)), ((), ())),
        preferred_element_type=jnp.float32)


def kernel(ti, tjs, ci, cjs, emb_weight):
    del ci
    hist = cjs.reshape(_L, _N)
    out2d = pl.pallas_call(
        _infectivity_body,
        grid=(_NB,),
        in_specs=[
            pl.BlockSpec(memory_space=pltpu.MemorySpace.VMEM),
            pl.BlockSpec(memory_space=pltpu.MemorySpace.VMEM),
            pl.BlockSpec(memory_space=pltpu.MemorySpace.HBM),
            pl.BlockSpec(memory_space=pltpu.MemorySpace.HBM),
        ],
        out_specs=pl.BlockSpec((_BM, _B), lambda i: (i, 0)),
        out_shape=jax.ShapeDtypeStruct((_N, _B), jnp.float32),
        scratch_shapes=[
            pltpu.VMEM((_B, _L), jnp.float32),
            pltpu.VMEM((_L, _N), jnp.int32),
            pltpu.VMEM((_L, _N), jnp.float32),
            pltpu.VMEM((_N, _N), jnp.float32),
            pltpu.SemaphoreType.DMA,
            pltpu.SemaphoreType.DMA((_NB,)),
        ],
    )(ti, tjs, hist, emb_weight)
    return out2d[:, :, None]


# speedup vs baseline: 1.1330x; 1.1330x over previous
import jax
import jax.numpy as jnp
from jax.experimental import pallas as pl
from jax.experimental.pallas import tpu as pltpu

_B = 1024
_L = 200
_N = 1000
_BM = 200
_NB = _N // _BM


def _infectivity_body(ti_ref, tjs_ref, hist_ref, emb_ref, out_hbm,
                      gt, hist_f, out_v, out_sems):
    gt[...] = jnp.exp(tjs_ref[...] - ti_ref[...])
    hist_f[...] = hist_ref[...].astype(jnp.float32)

    def out_cp(i):
        blk = pl.ds(i * _BM, _BM)
        return pltpu.make_async_copy(out_v.at[blk, :], out_hbm.at[blk, :],
                                     out_sems.at[i])

    for i in range(_NB):
        blk = pl.ds(i * _BM, _BM)
        a = jax.lax.dot_general(
            emb_ref[blk, :], hist_f[...], (((1,), (1,)), ((), ())),
            preferred_element_type=jnp.float32)
        out_v[blk, :] = jax.lax.dot_general(
            a, gt[...], (((1,), (1,)), ((), ())),
            preferred_element_type=jnp.float32)
        out_cp(i).start()

    for i in range(_NB):
        out_cp(i).wait()


def kernel(ti, tjs, ci, cjs, emb_weight):
    del ci
    hist = cjs.reshape(_L, _N)
    out2d = pl.pallas_call(
        _infectivity_body,
        in_specs=[
            pl.BlockSpec(memory_space=pltpu.MemorySpace.VMEM),
            pl.BlockSpec(memory_space=pltpu.MemorySpace.VMEM),
            pl.BlockSpec(memory_space=pltpu.MemorySpace.VMEM),
            pl.BlockSpec(memory_space=pltpu.MemorySpace.VMEM),
        ],
        out_specs=pl.BlockSpec(memory_space=pltpu.MemorySpace.HBM),
        out_shape=jax.ShapeDtypeStruct((_N, _B), jnp.float32),
        scratch_shapes=[
            pltpu.VMEM((_B, _L), jnp.float32),
            pltpu.VMEM((_L, _N), jnp.float32),
            pltpu.VMEM((_N, _B), jnp.float32),
            pltpu.SemaphoreType.DMA((_NB,)),
        ],
    )(ti, tjs, hist, emb_weight)
    return out2d[:, :, None]
